# baseline (speedup 1.0000x reference)
import functools

import jax
import jax.numpy as jnp
from jax import lax
from jax.experimental import pallas as pl
from jax.experimental.pallas import tpu as pltpu
from jax.experimental.pallas import tpu_sc as plsc

D = 8388608
N_PH = 64
N_MG = 1024

NC = 2
NS = 16
L = 16
NW = NC * NS
PER_W = D // NW
CHUNK = 8192
N_CHUNKS = PER_W // CHUNK
NBUF = 4
UNROLL = 8

_mesh = plsc.VectorSubcoreMesh(core_axis_name="c", subcore_axis_name="s")


@functools.partial(
    pl.kernel,
    mesh=_mesh,
    out_type=jax.ShapeDtypeStruct((D,), jnp.float32),
    compiler_params=pltpu.CompilerParams(
        needs_layout_passes=False, use_tc_tiling_on_sc=False),
    scratch_types=[
        pltpu.VMEM((N_PH,), jnp.float32),
        pltpu.VMEM((N_MG,), jnp.float32),
        pltpu.VMEM((N_PH * L,), jnp.float32),
        pltpu.VMEM((N_MG * L,), jnp.float32),
    ] + [pltpu.VMEM((CHUNK,), jnp.int32) for _ in range(2 * NBUF)]
      + [pltpu.VMEM((CHUNK,), jnp.float32) for _ in range(NBUF)]
      + [pltpu.SemaphoreType.DMA for _ in range(2 * NBUF)],
)
def _sc_lookup(pi_hbm, mi_hbm, pct_hbm, met_hbm, out_hbm,
               pct_v, met_v, pct_rep, met_rep, *bufs):
    pi_bufs = bufs[0:NBUF]
    mi_bufs = bufs[NBUF:2 * NBUF]
    o_bufs = bufs[2 * NBUF:3 * NBUF]
    sems_in = bufs[3 * NBUF:4 * NBUF]
    sems_out = bufs[4 * NBUF:5 * NBUF]

    wid = lax.axis_index("s") * NC + lax.axis_index("c")
    base = wid * PER_W

    def start_in(g, b):
        off = base + g * CHUNK
        pltpu.async_copy(pi_hbm.at[pl.ds(off, CHUNK)], pi_bufs[b], sems_in[b])
        pltpu.async_copy(mi_hbm.at[pl.ds(off, CHUNK)], mi_bufs[b], sems_in[b])

    def wait_in(b):
        pltpu.make_async_copy(pi_hbm.at[pl.ds(0, CHUNK)], pi_bufs[b],
                              sems_in[b]).wait()
        pltpu.make_async_copy(mi_hbm.at[pl.ds(0, CHUNK)], mi_bufs[b],
                              sems_in[b]).wait()

    def start_out(g, b):
        off = base + g * CHUNK
        pltpu.async_copy(o_bufs[b], out_hbm.at[pl.ds(off, CHUNK)], sems_out[b])

    def wait_out(b):
        pltpu.make_async_copy(o_bufs[b], out_hbm.at[pl.ds(0, CHUNK)],
                              sems_out[b]).wait()

    for g in range(NBUF):
        start_in(g, g)

    pltpu.sync_copy(pct_hbm, pct_v)
    pltpu.sync_copy(met_hbm, met_v)

    def compute(b):
        pi_buf, mi_buf, o_buf = pi_bufs[b], mi_bufs[b], o_bufs[b]

        @plsc.parallel_loop(0, CHUNK, L, unroll=UNROLL)
        def _(off):
            pidx = pi_buf[pl.ds(off, L)]
            midx = mi_buf[pl.ds(off, L)]
            cv = plsc.load_gather(pct_v, [pidx])
            mv = plsc.load_gather(met_v, [midx])
            o_buf[pl.ds(off, L)] = cv * mv

    n_pairs = N_CHUNKS // NBUF

    def pair_body(k, _):
        g0 = k * NBUF
        for b in range(NBUF):
            g = g0 + b
            wait_in(b)

            @pl.when(k > 0)
            def _():
                wait_out(b)

            compute(b)
            start_out(g, b)

            @pl.when(k < n_pairs - 1)
            def _():
                start_in(g + NBUF, b)

        return 0

    lax.fori_loop(0, n_pairs, pair_body, 0)
    for b in range(NBUF):
        wait_out(b)


def kernel(phase_indices, mag_indices, phase_cos_table, mag_exp_table):
    pi = phase_indices.astype(jnp.int32)
    mi = mag_indices.astype(jnp.int32)
    pct = phase_cos_table.astype(jnp.float32)
    met = mag_exp_table.astype(jnp.float32)
    return _sc_lookup(pi, mi, pct, met)


# speedup vs baseline: 1.0991x; 1.0991x over previous
import functools

import jax
import jax.numpy as jnp
from jax import lax
from jax.experimental import pallas as pl
from jax.experimental.pallas import tpu as pltpu
from jax.experimental.pallas import tpu_sc as plsc

D = 8388608
N_PH = 64
N_MG = 1024

NC = 2
NS = 16
L = 16
NW = NC * NS
PER_W = D // NW
CHUNK = 8192
N_CHUNKS = PER_W // CHUNK
NBUF = 4
UNROLL = 8

_mesh = plsc.VectorSubcoreMesh(core_axis_name="c", subcore_axis_name="s")


@functools.partial(
    pl.kernel,
    mesh=_mesh,
    out_type=jax.ShapeDtypeStruct((D,), jnp.float32),
    compiler_params=pltpu.CompilerParams(
        needs_layout_passes=False, use_tc_tiling_on_sc=False),
    scratch_types=[
        pltpu.VMEM((N_PH,), jnp.float32),
        pltpu.VMEM((N_MG,), jnp.float32),
        pltpu.VMEM((N_PH * L,), jnp.float32),
        pltpu.VMEM((N_MG * L,), jnp.float32),
    ] + [pltpu.VMEM((CHUNK,), jnp.int32) for _ in range(2 * NBUF)]
      + [pltpu.VMEM((CHUNK,), jnp.float32) for _ in range(NBUF)]
      + [pltpu.SemaphoreType.DMA for _ in range(2 * NBUF)],
)
def _sc_lookup(pi_hbm, mi_hbm, pct_hbm, met_hbm, out_hbm,
               pct_v, met_v, pct_rep, met_rep, *bufs):
    pi_bufs = bufs[0:NBUF]
    mi_bufs = bufs[NBUF:2 * NBUF]
    o_bufs = bufs[2 * NBUF:3 * NBUF]
    sems_in = bufs[3 * NBUF:4 * NBUF]
    sems_out = bufs[4 * NBUF:5 * NBUF]

    wid = lax.axis_index("s") * NC + lax.axis_index("c")
    base = wid * PER_W

    def start_in(g, b):
        off = base + g * CHUNK
        pltpu.async_copy(pi_hbm.at[pl.ds(off, CHUNK)], pi_bufs[b], sems_in[b])
        pltpu.async_copy(mi_hbm.at[pl.ds(off, CHUNK)], mi_bufs[b], sems_in[b])

    def wait_in(b):
        pltpu.make_async_copy(pi_hbm.at[pl.ds(0, CHUNK)], pi_bufs[b],
                              sems_in[b]).wait()
        pltpu.make_async_copy(mi_hbm.at[pl.ds(0, CHUNK)], mi_bufs[b],
                              sems_in[b]).wait()

    def start_out(g, b):
        off = base + g * CHUNK
        pltpu.async_copy(o_bufs[b], out_hbm.at[pl.ds(off, CHUNK)], sems_out[b])

    def wait_out(b):
        pltpu.make_async_copy(o_bufs[b], out_hbm.at[pl.ds(0, CHUNK)],
                              sems_out[b]).wait()

    for g in range(NBUF):
        start_in(g, g)

    pltpu.sync_copy(pct_hbm, pct_v)
    pltpu.sync_copy(met_hbm, met_v)

    lanes = lax.iota(jnp.int32, L)

    def build_rep(src_ref, rep_ref, n):
        @plsc.parallel_loop(0, n * L, L, unroll=8)
        def _(i):
            v = plsc.load_gather(src_ref, [jnp.full((L,), i >> 4, jnp.int32)])
            rep_ref[pl.ds(i, L)] = v

    build_rep(pct_v, pct_rep, N_PH)
    build_rep(met_v, met_rep, N_MG)

    def compute(b):
        pi_buf, mi_buf, o_buf = pi_bufs[b], mi_bufs[b], o_bufs[b]

        @plsc.parallel_loop(0, CHUNK, L, unroll=UNROLL)
        def _(off):
            pidx = pi_buf[pl.ds(off, L)]
            midx = mi_buf[pl.ds(off, L)]
            cv = plsc.load_gather(pct_rep, [(pidx << 4) | lanes])
            mv = plsc.load_gather(met_rep, [(midx << 4) | lanes])
            o_buf[pl.ds(off, L)] = cv * mv

    n_pairs = N_CHUNKS // NBUF

    def pair_body(k, _):
        g0 = k * NBUF
        for b in range(NBUF):
            g = g0 + b
            wait_in(b)

            @pl.when(k > 0)
            def _():
                wait_out(b)

            compute(b)
            start_out(g, b)

            @pl.when(k < n_pairs - 1)
            def _():
                start_in(g + NBUF, b)

        return 0

    lax.fori_loop(0, n_pairs, pair_body, 0)
    for b in range(NBUF):
        wait_out(b)


def kernel(phase_indices, mag_indices, phase_cos_table, mag_exp_table):
    pi = phase_indices.astype(jnp.int32)
    mi = mag_indices.astype(jnp.int32)
    pct = phase_cos_table.astype(jnp.float32)
    met = mag_exp_table.astype(jnp.float32)
    return _sc_lookup(pi, mi, pct, met)
